# baseline (speedup 1.0000x reference)
import functools

import jax
import jax.numpy as jnp
from jax import lax
from jax.experimental import pallas as pl
from jax.experimental.pallas import tpu as pltpu
from jax.experimental.pallas import tpu_sc as plsc

_B = 8
_K = 2048
_D = 3
_L = 16
_NW = 32
_WPB = _NW // _B
_QPW = _K // _WPB
_R2 = 1.1 * 1.1
_MIN_CNT = 2.5

_GATHER_DNUMS = lax.GatherDimensionNumbers(
    offset_dims=(), collapsed_slice_dims=(0,), start_index_map=(0,))


def _bcast_lane(v, u):
    idx = jnp.full((_L, 1), u, jnp.int32)
    return lax.gather(v, idx, _GATHER_DNUMS, (1,),
                      mode=lax.GatherScatterMode.PROMISE_IN_BOUNDS)


def _bf16_round(v):
    u = lax.bitcast_convert_type(v, jnp.uint32)
    lsb = lax.shift_right_logical(u, jnp.uint32(16)) & jnp.uint32(1)
    r = (u + jnp.uint32(0x7FFF) + lsb) & jnp.uint32(0xFFFF0000)
    return lax.bitcast_convert_type(r, jnp.float32)


def _ror_body(xt_hbm, out_hbm, keys_v, keysb_v, sqk_v, out_v):
    c = lax.axis_index("c")
    s = lax.axis_index("s")
    wid = s * 2 + c
    b = wid // _WPB
    qq = wid % _WPB
    qbase = qq * _QPW

    pltpu.sync_copy(xt_hbm.at[b], keys_v)

    def sq_body(j, _):
        off = j * _L
        kx = keys_v[0, pl.ds(off, _L)]
        ky = keys_v[1, pl.ds(off, _L)]
        kz = keys_v[2, pl.ds(off, _L)]
        sqk_v[pl.ds(off, _L)] = (kx * kx + ky * ky) + kz * kz
        keysb_v[0, pl.ds(off, _L)] = _bf16_round(kx)
        keysb_v[1, pl.ds(off, _L)] = _bf16_round(ky)
        keysb_v[2, pl.ds(off, _L)] = _bf16_round(kz)
        return 0

    lax.fori_loop(0, _K // _L, sq_body, 0, unroll=False)

    def q_body(i, _):
        qoff = qbase + i * _L
        qx = keys_v[0, pl.ds(qoff, _L)]
        qy = keys_v[1, pl.ds(qoff, _L)]
        qz = keys_v[2, pl.ds(qoff, _L)]
        qxb = keysb_v[0, pl.ds(qoff, _L)]
        qyb = keysb_v[1, pl.ds(qoff, _L)]
        qzb = keysb_v[2, pl.ds(qoff, _L)]
        sqq = sqk_v[pl.ds(qoff, _L)]

        def k_body(j, cnt):
            off = j * _L
            kxv = keysb_v[0, pl.ds(off, _L)]
            kyv = keysb_v[1, pl.ds(off, _L)]
            kzv = keysb_v[2, pl.ds(off, _L)]
            sqv = sqk_v[pl.ds(off, _L)]
            for u in range(_L):
                kx = _bcast_lane(kxv, u)
                ky = _bcast_lane(kyv, u)
                kz = _bcast_lane(kzv, u)
                sqk = _bcast_lane(sqv, u)
                p = (qxb * kx + qyb * ky) + qzb * kz
                d2 = (sqq + sqk) - 2.0 * p
                cnt = cnt + jnp.where(d2 <= _R2, 1.0, 0.0)
            return cnt

        cnt = lax.fori_loop(0, _K // _L, k_body,
                            jnp.zeros((_L,), jnp.float32), unroll=False)
        keep = cnt >= _MIN_CNT
        ooff = i * _L
        out_v[0, pl.ds(ooff, _L)] = jnp.where(keep, qx, 0.0)
        out_v[1, pl.ds(ooff, _L)] = jnp.where(keep, qy, 0.0)
        out_v[2, pl.ds(ooff, _L)] = jnp.where(keep, qz, 0.0)
        return 0

    lax.fori_loop(0, _QPW // _L, q_body, 0, unroll=False)

    pltpu.sync_copy(out_v, out_hbm.at[b, qq])


@jax.jit
def kernel(x):
    xt = jnp.transpose(x, (0, 2, 1))
    run = functools.partial(
        pl.kernel,
        out_type=jax.ShapeDtypeStruct((_B, _WPB, _D, _QPW), jnp.float32),
        mesh=plsc.VectorSubcoreMesh(core_axis_name="c", subcore_axis_name="s"),
        scratch_types=[
            pltpu.VMEM((_D, _K), jnp.float32),
            pltpu.VMEM((_D, _K), jnp.float32),
            pltpu.VMEM((_K,), jnp.float32),
            pltpu.VMEM((_D, _QPW), jnp.float32),
        ],
    )(_ror_body)
    out = run(xt)
    return jnp.transpose(out, (0, 1, 3, 2)).reshape(_B, _K, _D)


# speedup vs baseline: 1.0063x; 1.0063x over previous
import functools

import jax
import jax.numpy as jnp
from jax import lax
from jax.experimental import pallas as pl
from jax.experimental.pallas import tpu as pltpu
from jax.experimental.pallas import tpu_sc as plsc

_B = 8
_K = 2048
_D = 3
_L = 16
_NW = 32
_WPB = _NW // _B
_QPW = _K // _WPB
_R2 = 1.1 * 1.1
_MIN_CNT = 2.5


def _bf16_round(v):
    u = lax.bitcast_convert_type(v, jnp.uint32)
    lsb = lax.shift_right_logical(u, jnp.uint32(16)) & jnp.uint32(1)
    r = (u + jnp.uint32(0x7FFF) + lsb) & jnp.uint32(0xFFFF0000)
    return lax.bitcast_convert_type(r, jnp.float32)


def _ror_body(xt_hbm, out_hbm, keys_v, keysb_v, sqk_v, out_v):
    c = lax.axis_index("c")
    s = lax.axis_index("s")
    wid = s * 2 + c
    b = wid // _WPB
    qq = wid % _WPB
    qbase = qq * _QPW

    pltpu.sync_copy(xt_hbm.at[b], keys_v)

    def sq_body(j, _):
        off = j * _L
        kx = keys_v[0, pl.ds(off, _L)]
        ky = keys_v[1, pl.ds(off, _L)]
        kz = keys_v[2, pl.ds(off, _L)]
        sqk_v[pl.ds(off, _L)] = (kx * kx + ky * ky) + kz * kz
        keysb_v[0, pl.ds(off, _L)] = _bf16_round(kx)
        keysb_v[1, pl.ds(off, _L)] = _bf16_round(ky)
        keysb_v[2, pl.ds(off, _L)] = _bf16_round(kz)
        return 0

    lax.fori_loop(0, _K // _L, sq_body, 0, unroll=False)

    def q_body(i, _):
        qoff = qbase + i * _L
        qx = keys_v[0, pl.ds(qoff, _L)]
        qy = keys_v[1, pl.ds(qoff, _L)]
        qz = keys_v[2, pl.ds(qoff, _L)]
        qxb = keysb_v[0, pl.ds(qoff, _L)]
        qyb = keysb_v[1, pl.ds(qoff, _L)]
        qzb = keysb_v[2, pl.ds(qoff, _L)]
        sqq = sqk_v[pl.ds(qoff, _L)]

        def k_body(j, cnt):
            off = j * _L
            kxv = keysb_v[0, pl.ds(off, _L)]
            kyv = keysb_v[1, pl.ds(off, _L)]
            kzv = keysb_v[2, pl.ds(off, _L)]
            sqv = sqk_v[pl.ds(off, _L)]
            for u in range(_L):
                kx = jnp.full((_L,), kxv[u], jnp.float32)
                ky = jnp.full((_L,), kyv[u], jnp.float32)
                kz = jnp.full((_L,), kzv[u], jnp.float32)
                sqk = jnp.full((_L,), sqv[u], jnp.float32)
                p = (qxb * kx + qyb * ky) + qzb * kz
                d2 = (sqq + sqk) - 2.0 * p
                cnt = cnt + jnp.where(d2 <= _R2, 1.0, 0.0)
            return cnt

        cnt = lax.fori_loop(0, _K // _L, k_body,
                            jnp.zeros((_L,), jnp.float32), unroll=False)
        keep = cnt >= _MIN_CNT
        ooff = i * _L
        out_v[0, pl.ds(ooff, _L)] = jnp.where(keep, qx, 0.0)
        out_v[1, pl.ds(ooff, _L)] = jnp.where(keep, qy, 0.0)
        out_v[2, pl.ds(ooff, _L)] = jnp.where(keep, qz, 0.0)
        return 0

    lax.fori_loop(0, _QPW // _L, q_body, 0, unroll=False)

    pltpu.sync_copy(out_v, out_hbm.at[b, qq])


@jax.jit
def kernel(x):
    xt = jnp.transpose(x, (0, 2, 1))
    run = functools.partial(
        pl.kernel,
        out_type=jax.ShapeDtypeStruct((_B, _WPB, _D, _QPW), jnp.float32),
        mesh=plsc.VectorSubcoreMesh(core_axis_name="c", subcore_axis_name="s"),
        scratch_types=[
            pltpu.VMEM((_D, _K), jnp.float32),
            pltpu.VMEM((_D, _K), jnp.float32),
            pltpu.VMEM((_K,), jnp.float32),
            pltpu.VMEM((_D, _QPW), jnp.float32),
        ],
    )(_ror_body)
    out = run(xt)
    return jnp.transpose(out, (0, 1, 3, 2)).reshape(_B, _K, _D)


# speedup vs baseline: 1.6388x; 1.6285x over previous
import functools

import jax
import jax.numpy as jnp
from jax import lax
from jax.experimental import pallas as pl
from jax.experimental.pallas import tpu as pltpu
from jax.experimental.pallas import tpu_sc as plsc

_B = 8
_K = 2048
_D = 3
_L = 16
_NW = 32
_WPB = _NW // _B
_QPW = _K // _WPB
_R2 = 1.1 * 1.1
_MIN_CNT = 2.5
_GPB = 4
_NBLK = _K // (_L * _GPB)


def _bf16_round(v):
    u = lax.bitcast_convert_type(v, jnp.uint32)
    lsb = lax.shift_right_logical(u, jnp.uint32(16)) & jnp.uint32(1)
    r = (u + jnp.uint32(0x7FFF) + lsb) & jnp.uint32(0xFFFF0000)
    return lax.bitcast_convert_type(r, jnp.float32)


def _ror_body(xt_hbm, out_hbm, keys_v, keysb_v, sqk_v, out_v, cnt_v):
    c = lax.axis_index("c")
    s = lax.axis_index("s")
    wid = s * 2 + c
    b = wid // _WPB
    qq = wid % _WPB
    qbase = qq * _QPW

    pltpu.sync_copy(xt_hbm.at[b], keys_v)

    def sq_body(j, _):
        off = j * _L
        kx = keys_v[0, pl.ds(off, _L)]
        ky = keys_v[1, pl.ds(off, _L)]
        kz = keys_v[2, pl.ds(off, _L)]
        sqk_v[pl.ds(off, _L)] = (kx * kx + ky * ky) + kz * kz
        keysb_v[0, pl.ds(off, _L)] = _bf16_round(kx)
        keysb_v[1, pl.ds(off, _L)] = _bf16_round(ky)
        keysb_v[2, pl.ds(off, _L)] = _bf16_round(kz)
        return 0

    lax.fori_loop(0, _K // _L, sq_body, 0, unroll=False)

    def q_body(i, _):
        qoff = qbase + i * _L
        qx = keys_v[0, pl.ds(qoff, _L)]
        qy = keys_v[1, pl.ds(qoff, _L)]
        qz = keys_v[2, pl.ds(qoff, _L)]
        qxb = keysb_v[0, pl.ds(qoff, _L)]
        qyb = keysb_v[1, pl.ds(qoff, _L)]
        qzb = keysb_v[2, pl.ds(qoff, _L)]
        sqq = sqk_v[pl.ds(qoff, _L)]

        def k_body(j, cnt):
            off = j * _L
            kxv = keysb_v[0, pl.ds(off, _L)]
            kyv = keysb_v[1, pl.ds(off, _L)]
            kzv = keysb_v[2, pl.ds(off, _L)]
            sqv = sqk_v[pl.ds(off, _L)]
            for u in range(_L):
                kx = jnp.full((_L,), kxv[u], jnp.float32)
                ky = jnp.full((_L,), kyv[u], jnp.float32)
                kz = jnp.full((_L,), kzv[u], jnp.float32)
                sqk = jnp.full((_L,), sqv[u], jnp.float32)
                p = (qxb * kx + qyb * ky) + qzb * kz
                d2 = (sqq + sqk) - 2.0 * p
                cnt = cnt + jnp.where(d2 <= _R2, 1.0, 0.0)
            return cnt

        cnt_v[...] = jnp.zeros((_L,), jnp.float32)

        def blk_body(t, _):
            cnt0 = cnt_v[...]
            m = [cnt0[u] for u in range(_L)]
            while len(m) > 1:
                m = [jnp.minimum(m[v], m[v + 1]) for v in range(0, len(m), 2)]

            @pl.when(m[0] < _MIN_CNT)
            def _scan():
                cnt_v[...] = lax.fori_loop(t * _GPB, (t + 1) * _GPB,
                                           k_body, cnt0, unroll=False)

            return 0

        lax.fori_loop(0, _NBLK, blk_body, 0, unroll=False)
        keep = cnt_v[...] >= _MIN_CNT
        ooff = i * _L
        out_v[0, pl.ds(ooff, _L)] = jnp.where(keep, qx, 0.0)
        out_v[1, pl.ds(ooff, _L)] = jnp.where(keep, qy, 0.0)
        out_v[2, pl.ds(ooff, _L)] = jnp.where(keep, qz, 0.0)
        return 0

    lax.fori_loop(0, _QPW // _L, q_body, 0, unroll=False)

    pltpu.sync_copy(out_v, out_hbm.at[b, qq])


@jax.jit
def kernel(x):
    xt = jnp.transpose(x, (0, 2, 1))
    run = functools.partial(
        pl.kernel,
        out_type=jax.ShapeDtypeStruct((_B, _WPB, _D, _QPW), jnp.float32),
        mesh=plsc.VectorSubcoreMesh(core_axis_name="c", subcore_axis_name="s"),
        scratch_types=[
            pltpu.VMEM((_D, _K), jnp.float32),
            pltpu.VMEM((_D, _K), jnp.float32),
            pltpu.VMEM((_K,), jnp.float32),
            pltpu.VMEM((_D, _QPW), jnp.float32),
            pltpu.VMEM((_L,), jnp.float32),
        ],
    )(_ror_body)
    out = run(xt)
    return jnp.transpose(out, (0, 1, 3, 2)).reshape(_B, _K, _D)


# speedup vs baseline: 1.7433x; 1.0638x over previous
import functools

import jax
import jax.numpy as jnp
from jax import lax
from jax.experimental import pallas as pl
from jax.experimental.pallas import tpu as pltpu
from jax.experimental.pallas import tpu_sc as plsc

_B = 8
_K = 2048
_D = 3
_L = 16
_NW = 32
_WPB = _NW // _B
_QPW = _K // _WPB
_R2 = 1.1 * 1.1
_MIN_CNT = 2.5
_GPB = 4
_NBLK = _K // (_L * _GPB)


_GATHER_DNUMS = lax.GatherDimensionNumbers(
    offset_dims=(), collapsed_slice_dims=(0,), start_index_map=(0,))


def _lane_min(v):
    lane = lax.iota(jnp.int32, _L)
    s = v
    for sh in (8, 4, 2, 1):
        idx = (lane + sh) & (_L - 1)
        s = jnp.minimum(s, lax.gather(s, idx[:, None], _GATHER_DNUMS, (1,),
                                      mode=lax.GatherScatterMode.PROMISE_IN_BOUNDS))
    return s


def _bf16_round(v):
    u = lax.bitcast_convert_type(v, jnp.uint32)
    lsb = lax.shift_right_logical(u, jnp.uint32(16)) & jnp.uint32(1)
    r = (u + jnp.uint32(0x7FFF) + lsb) & jnp.uint32(0xFFFF0000)
    return lax.bitcast_convert_type(r, jnp.float32)


def _ror_body(xt_hbm, out_hbm, keys_v, keysb_v, sqk_v, out_v, cnt_v):
    c = lax.axis_index("c")
    s = lax.axis_index("s")
    wid = s * 2 + c
    b = wid // _WPB
    qq = wid % _WPB
    qbase = qq * _QPW

    pltpu.sync_copy(xt_hbm.at[b], keys_v)

    def sq_body(j, _):
        off = j * _L
        kx = keys_v[0, pl.ds(off, _L)]
        ky = keys_v[1, pl.ds(off, _L)]
        kz = keys_v[2, pl.ds(off, _L)]
        sqk_v[pl.ds(off, _L)] = (kx * kx + ky * ky) + kz * kz
        keysb_v[0, pl.ds(off, _L)] = _bf16_round(kx)
        keysb_v[1, pl.ds(off, _L)] = _bf16_round(ky)
        keysb_v[2, pl.ds(off, _L)] = _bf16_round(kz)
        return 0

    lax.fori_loop(0, _K // _L, sq_body, 0, unroll=False)

    def q_body(i, _):
        qoff = qbase + i * _L
        qx = keys_v[0, pl.ds(qoff, _L)]
        qy = keys_v[1, pl.ds(qoff, _L)]
        qz = keys_v[2, pl.ds(qoff, _L)]
        qxm = -2.0 * keysb_v[0, pl.ds(qoff, _L)]
        qym = -2.0 * keysb_v[1, pl.ds(qoff, _L)]
        qzm = -2.0 * keysb_v[2, pl.ds(qoff, _L)]
        sqq = sqk_v[pl.ds(qoff, _L)]

        def k_body(j, cnt):
            off = j * _L
            kxv = keysb_v[0, pl.ds(off, _L)]
            kyv = keysb_v[1, pl.ds(off, _L)]
            kzv = keysb_v[2, pl.ds(off, _L)]
            sqv = sqk_v[pl.ds(off, _L)]
            for u in range(_L):
                kx = jnp.full((_L,), kxv[u], jnp.float32)
                ky = jnp.full((_L,), kyv[u], jnp.float32)
                kz = jnp.full((_L,), kzv[u], jnp.float32)
                sqk = jnp.full((_L,), sqv[u], jnp.float32)
                p2 = (qxm * kx + qym * ky) + qzm * kz
                d2 = (sqq + sqk) + p2
                cnt = cnt + jnp.where(d2 <= _R2, 1.0, 0.0)
            return cnt

        cnt_v[...] = jnp.zeros((_L,), jnp.float32)

        def blk_body(t, _):
            cnt0 = cnt_v[...]

            @pl.when(_lane_min(cnt0)[0] < _MIN_CNT)
            def _scan():
                cnt_v[...] = lax.fori_loop(t * _GPB, (t + 1) * _GPB,
                                           k_body, cnt0, unroll=False)

            return 0

        lax.fori_loop(0, _NBLK, blk_body, 0, unroll=False)
        keep = cnt_v[...] >= _MIN_CNT
        ooff = i * _L
        out_v[0, pl.ds(ooff, _L)] = jnp.where(keep, qx, 0.0)
        out_v[1, pl.ds(ooff, _L)] = jnp.where(keep, qy, 0.0)
        out_v[2, pl.ds(ooff, _L)] = jnp.where(keep, qz, 0.0)
        return 0

    lax.fori_loop(0, _QPW // _L, q_body, 0, unroll=False)

    pltpu.sync_copy(out_v, out_hbm.at[b, qq])


@jax.jit
def kernel(x):
    xt = jnp.transpose(x, (0, 2, 1))
    run = functools.partial(
        pl.kernel,
        out_type=jax.ShapeDtypeStruct((_B, _WPB, _D, _QPW), jnp.float32),
        mesh=plsc.VectorSubcoreMesh(core_axis_name="c", subcore_axis_name="s"),
        scratch_types=[
            pltpu.VMEM((_D, _K), jnp.float32),
            pltpu.VMEM((_D, _K), jnp.float32),
            pltpu.VMEM((_K,), jnp.float32),
            pltpu.VMEM((_D, _QPW), jnp.float32),
            pltpu.VMEM((_L,), jnp.float32),
        ],
    )(_ror_body)
    out = run(xt)
    return jnp.transpose(out, (0, 1, 3, 2)).reshape(_B, _K, _D)


# speedup vs baseline: 1.7444x; 1.0006x over previous
import functools

import jax
import jax.numpy as jnp
from jax import lax
from jax.experimental import pallas as pl
from jax.experimental.pallas import tpu as pltpu
from jax.experimental.pallas import tpu_sc as plsc

_B = 8
_K = 2048
_D = 3
_L = 16
_NW = 32
_WPB = _NW // _B
_QPW = _K // _WPB
_R2 = 1.1 * 1.1
_MIN_CNT = 2.5
_GPB = 4
_NBLK = _K // (_L * _GPB)


_GATHER_DNUMS = lax.GatherDimensionNumbers(
    offset_dims=(), collapsed_slice_dims=(0,), start_index_map=(0,))


def _lane_min(v):
    lane = lax.iota(jnp.int32, _L)
    s = v
    for sh in (8, 4, 2, 1):
        idx = (lane + sh) & (_L - 1)
        s = jnp.minimum(s, lax.gather(s, idx[:, None], _GATHER_DNUMS, (1,),
                                      mode=lax.GatherScatterMode.PROMISE_IN_BOUNDS))
    return s


def _bf16_round(v):
    u = lax.bitcast_convert_type(v, jnp.uint32)
    lsb = lax.shift_right_logical(u, jnp.uint32(16)) & jnp.uint32(1)
    r = (u + jnp.uint32(0x7FFF) + lsb) & jnp.uint32(0xFFFF0000)
    return lax.bitcast_convert_type(r, jnp.float32)


def _ror_body(xt_hbm, out_hbm, keys_v, keysb_v, sqk_v, out_v, cnt_v):
    c = lax.axis_index("c")
    s = lax.axis_index("s")
    wid = s * 2 + c
    b = wid // _WPB
    qq = wid % _WPB

    pltpu.sync_copy(xt_hbm.at[b], keys_v)

    def sq_body(j, _):
        off = j * _L
        kx = keys_v[0, pl.ds(off, _L)]
        ky = keys_v[1, pl.ds(off, _L)]
        kz = keys_v[2, pl.ds(off, _L)]
        sqk_v[pl.ds(off, _L)] = (kx * kx + ky * ky) + kz * kz
        keysb_v[0, pl.ds(off, _L)] = _bf16_round(kx)
        keysb_v[1, pl.ds(off, _L)] = _bf16_round(ky)
        keysb_v[2, pl.ds(off, _L)] = _bf16_round(kz)
        return 0

    lax.fori_loop(0, _K // _L, sq_body, 0, unroll=False)

    def q_body(i, _):
        v = qq + _WPB * i
        qoff = v * _L
        qx = keys_v[0, pl.ds(qoff, _L)]
        qy = keys_v[1, pl.ds(qoff, _L)]
        qz = keys_v[2, pl.ds(qoff, _L)]
        qxm = -2.0 * keysb_v[0, pl.ds(qoff, _L)]
        qym = -2.0 * keysb_v[1, pl.ds(qoff, _L)]
        qzm = -2.0 * keysb_v[2, pl.ds(qoff, _L)]
        sqq = sqk_v[pl.ds(qoff, _L)]

        def k_body(j, cnt):
            off = ((v + j) & (_K // _L - 1)) * _L
            kxv = keysb_v[0, pl.ds(off, _L)]
            kyv = keysb_v[1, pl.ds(off, _L)]
            kzv = keysb_v[2, pl.ds(off, _L)]
            sqv = sqk_v[pl.ds(off, _L)]
            for u in range(_L):
                kx = jnp.full((_L,), kxv[u], jnp.float32)
                ky = jnp.full((_L,), kyv[u], jnp.float32)
                kz = jnp.full((_L,), kzv[u], jnp.float32)
                sqk = jnp.full((_L,), sqv[u], jnp.float32)
                p2 = (qxm * kx + qym * ky) + qzm * kz
                d2 = (sqq + sqk) + p2
                cnt = cnt + jnp.where(d2 <= _R2, 1.0, 0.0)
            return cnt

        cnt_v[...] = jnp.zeros((_L,), jnp.float32)

        def blk_body(t, _):
            cnt0 = cnt_v[...]

            @pl.when(_lane_min(cnt0)[0] < _MIN_CNT)
            def _scan():
                cnt_v[...] = lax.fori_loop(t * _GPB, (t + 1) * _GPB,
                                           k_body, cnt0, unroll=False)

            return 0

        lax.fori_loop(0, _NBLK, blk_body, 0, unroll=False)
        keep = cnt_v[...] >= _MIN_CNT
        ooff = i * _L
        out_v[0, pl.ds(ooff, _L)] = jnp.where(keep, qx, 0.0)
        out_v[1, pl.ds(ooff, _L)] = jnp.where(keep, qy, 0.0)
        out_v[2, pl.ds(ooff, _L)] = jnp.where(keep, qz, 0.0)
        return 0

    lax.fori_loop(0, _QPW // _L, q_body, 0, unroll=False)

    pltpu.sync_copy(out_v, out_hbm.at[b, qq])


@jax.jit
def kernel(x):
    nrm = jnp.sum(x * x, axis=-1)
    order = jnp.argsort(nrm, axis=1)
    xs = jnp.take_along_axis(x, order[:, :, None], axis=1)
    xt = jnp.transpose(xs, (0, 2, 1))
    run = functools.partial(
        pl.kernel,
        out_type=jax.ShapeDtypeStruct((_B, _WPB, _D, _QPW), jnp.float32),
        mesh=plsc.VectorSubcoreMesh(core_axis_name="c", subcore_axis_name="s"),
        scratch_types=[
            pltpu.VMEM((_D, _K), jnp.float32),
            pltpu.VMEM((_D, _K), jnp.float32),
            pltpu.VMEM((_K,), jnp.float32),
            pltpu.VMEM((_D, _QPW), jnp.float32),
            pltpu.VMEM((_L,), jnp.float32),
        ],
    )(_ror_body)
    out = run(xt)
    o = jnp.transpose(out, (0, 1, 3, 2))
    o = o.reshape(_B, _WPB, _QPW // _L, _L, _D)
    o = jnp.transpose(o, (0, 2, 1, 3, 4)).reshape(_B, _K, _D)
    inv = jnp.zeros((_B, _K), jnp.int32).at[
        jnp.arange(_B)[:, None], order].set(jnp.arange(_K, dtype=jnp.int32)[None, :])
    return jnp.take_along_axis(o, inv[:, :, None], axis=1)


# speedup vs baseline: 2.0857x; 1.1956x over previous
import functools

import jax
import jax.numpy as jnp
from jax import lax
from jax.experimental import pallas as pl
from jax.experimental.pallas import tpu as pltpu
from jax.experimental.pallas import tpu_sc as plsc

_B = 8
_K = 2048
_D = 3
_L = 16
_NW = 32
_WPB = _NW // _B
_QPW = _K // _WPB
_R2 = 1.1 * 1.1
_MIN_CNT = 2.5
_GPB = 4
_NBLK = _K // (_L * _GPB)


_GATHER_DNUMS = lax.GatherDimensionNumbers(
    offset_dims=(), collapsed_slice_dims=(0,), start_index_map=(0,))


def _lane_min(v):
    lane = lax.iota(jnp.int32, _L)
    s = v
    for sh in (8, 4, 2, 1):
        idx = (lane + sh) & (_L - 1)
        s = jnp.minimum(s, lax.gather(s, idx[:, None], _GATHER_DNUMS, (1,),
                                      mode=lax.GatherScatterMode.PROMISE_IN_BOUNDS))
    return s


def _bf16_round(v):
    u = lax.bitcast_convert_type(v, jnp.uint32)
    lsb = lax.shift_right_logical(u, jnp.uint32(16)) & jnp.uint32(1)
    r = (u + jnp.uint32(0x7FFF) + lsb) & jnp.uint32(0xFFFF0000)
    return lax.bitcast_convert_type(r, jnp.float32)


def _ror_body(xt_hbm, out_hbm, keys_v, keysb_v, sqk_v, out_v, cnt_v):
    c = lax.axis_index("c")
    s = lax.axis_index("s")
    wid = s * 2 + c
    b = wid // _WPB
    qq = wid % _WPB

    pltpu.sync_copy(xt_hbm.at[b], keys_v)

    def sq_body(j, _):
        off = j * _L
        kx = keys_v[0, pl.ds(off, _L)]
        ky = keys_v[1, pl.ds(off, _L)]
        kz = keys_v[2, pl.ds(off, _L)]
        sqk_v[pl.ds(off, _L)] = (kx * kx + ky * ky) + kz * kz
        keysb_v[0, pl.ds(off, _L)] = _bf16_round(kx)
        keysb_v[1, pl.ds(off, _L)] = _bf16_round(ky)
        keysb_v[2, pl.ds(off, _L)] = _bf16_round(kz)
        return 0

    lax.fori_loop(0, _K // _L, sq_body, 0, unroll=False)

    def q_body(i, _):
        v = qq + _WPB * i
        qoff = v * _L
        qxm = -2.0 * keysb_v[0, pl.ds(qoff, _L)]
        qym = -2.0 * keysb_v[1, pl.ds(qoff, _L)]
        qzm = -2.0 * keysb_v[2, pl.ds(qoff, _L)]
        sqq = sqk_v[pl.ds(qoff, _L)]

        def k_body(j, cnt):
            off = ((v + j) & (_K // _L - 1)) * _L
            kxv = keysb_v[0, pl.ds(off, _L)]
            kyv = keysb_v[1, pl.ds(off, _L)]
            kzv = keysb_v[2, pl.ds(off, _L)]
            sqv = sqk_v[pl.ds(off, _L)]
            for u in range(_L):
                kx = jnp.full((_L,), kxv[u], jnp.float32)
                ky = jnp.full((_L,), kyv[u], jnp.float32)
                kz = jnp.full((_L,), kzv[u], jnp.float32)
                sqk = jnp.full((_L,), sqv[u], jnp.float32)
                p2 = (qxm * kx + qym * ky) + qzm * kz
                d2 = (sqq + sqk) + p2
                cnt = cnt + jnp.where(d2 <= _R2, 1.0, 0.0)
            return cnt

        cnt_v[...] = jnp.zeros((_L,), jnp.float32)

        def blk_body(t, _):
            cnt0 = cnt_v[...]

            @pl.when(_lane_min(cnt0)[0] < _MIN_CNT)
            def _scan():
                cnt_v[...] = lax.fori_loop(t * _GPB, (t + 1) * _GPB,
                                           k_body, cnt0, unroll=False)

            return 0

        lax.fori_loop(0, _NBLK, blk_body, 0, unroll=False)
        keep = cnt_v[...] >= _MIN_CNT
        out_v[pl.ds(i * _L, _L)] = jnp.where(keep, 1.0, 0.0)
        return 0

    lax.fori_loop(0, _QPW // _L, q_body, 0, unroll=False)

    pltpu.sync_copy(out_v, out_hbm.at[b, qq])


@jax.jit
def kernel(x):
    nrm = jnp.sum(x * x, axis=-1)
    order = jnp.argsort(nrm, axis=1)
    xs = jnp.take_along_axis(x, order[:, :, None], axis=1)
    xt = jnp.transpose(xs, (0, 2, 1))
    run = functools.partial(
        pl.kernel,
        out_type=jax.ShapeDtypeStruct((_B, _WPB, _QPW), jnp.float32),
        mesh=plsc.VectorSubcoreMesh(core_axis_name="c", subcore_axis_name="s"),
        scratch_types=[
            pltpu.VMEM((_D, _K), jnp.float32),
            pltpu.VMEM((_D, _K), jnp.float32),
            pltpu.VMEM((_K,), jnp.float32),
            pltpu.VMEM((_QPW,), jnp.float32),
            pltpu.VMEM((_L,), jnp.float32),
        ],
    )(_ror_body)
    mask = run(xt)
    m = mask.reshape(_B, _WPB, _QPW // _L, _L)
    m = jnp.transpose(m, (0, 2, 1, 3)).reshape(_B, _K)
    rank = jnp.argsort(order, axis=1)
    m = jnp.take_along_axis(m, rank, axis=1)
    return x * m[:, :, None]
